# baseline (speedup 1.0000x reference)
import functools

import jax
import jax.numpy as jnp
from jax import lax
from jax.experimental import pallas as pl
from jax.experimental.pallas import tpu as pltpu
from jax.experimental.pallas import tpu_sc as plsc

N = 320000
F_IN = 128
H = 128
C = 64
B = 10000

RE = 6400
NE = N // RE
NW = 32
TR = N // NW
CR = 80
NCHUNK = TR // CR
STRIPE = 640
NSUB = STRIPE // CR


def _embed_body(x_ref, sc_ref, w1_ref, out_ref):
    x = x_ref[...].astype(jnp.bfloat16)
    h = jnp.maximum(
        jnp.dot(x, w1_ref[...].astype(jnp.bfloat16),
                preferred_element_type=jnp.float32), 0.0)
    out_ref[...] = h * sc_ref[...]


def _embed(X, ppr_scores, W1):
    return pl.pallas_call(
        _embed_body,
        grid=(NE,),
        in_specs=[
            pl.BlockSpec((RE, F_IN), lambda i: (i, 0)),
            pl.BlockSpec((RE, 1), lambda i: (i, 0)),
            pl.BlockSpec((F_IN, H), lambda i: (0, 0)),
        ],
        out_specs=pl.BlockSpec((RE, H), lambda i: (i, 0)),
        out_shape=jax.ShapeDtypeStruct((N, H), jnp.float32),
        compiler_params=pltpu.CompilerParams(
            dimension_semantics=("parallel",),
        ),
    )(X, ppr_scores.reshape(N, 1), W1)


def _sc_segsum(E, idx):
    mesh = plsc.VectorSubcoreMesh(core_axis_name="c", subcore_axis_name="s")

    @functools.partial(
        pl.kernel,
        mesh=mesh,
        out_type=jax.ShapeDtypeStruct((2, B, H), jnp.float32),
        scratch_types=[
            pltpu.VMEM((CR, H), jnp.float32),
            pltpu.VMEM((CR,), jnp.int32),
            pltpu.VMEM_SHARED((B, H), jnp.float32),
        ],
    )
    def k(e_hbm, idx_hbm, out_hbm, rows_v, idx_v, acc_sh):
        cid = lax.axis_index("c")
        sid = lax.axis_index("s")
        wid = cid * 16 + sid
        base = wid * TR

        zv = jnp.zeros((16,), jnp.float32)

        def z1(i, carry):
            def z2(j, carry2):
                rows_v[i, pl.ds(j * 16, 16)] = zv
                return carry2
            return lax.fori_loop(0, H // 16, z2, carry)

        lax.fori_loop(0, CR, z1, 0)
        for j in range(NSUB):
            st = sid * STRIPE + j * CR

            @pl.when(st + CR <= B)
            def _z(st=st):
                pltpu.sync_copy(rows_v, acc_sh.at[pl.ds(st, CR)])

        plsc.subcore_barrier()

        def body(kk, carry):
            b = base + kk * CR
            pltpu.sync_copy(e_hbm.at[pl.ds(b, CR)], rows_v)
            pltpu.sync_copy(idx_hbm.at[pl.ds(b, CR)], idx_v)
            pltpu.sync_copy(rows_v, acc_sh.at[idx_v], add=True)
            return carry

        lax.fori_loop(0, NCHUNK, body, 0)
        plsc.subcore_barrier()

        for j in range(NSUB):
            st = sid * STRIPE + j * CR

            @pl.when(st + CR <= B)
            def _wb(st=st):
                pltpu.sync_copy(acc_sh.at[pl.ds(st, CR)], rows_v)
                pltpu.sync_copy(rows_v, out_hbm.at[cid, pl.ds(st, CR)])

    return k(E, idx)


def _logits_body(p_ref, w2_ref, w3_ref, w4_ref, out_ref):
    p = (p_ref[0] + p_ref[1]).astype(jnp.bfloat16)
    w23 = jnp.dot(w2_ref[...].astype(jnp.bfloat16),
                  w3_ref[...].astype(jnp.bfloat16),
                  preferred_element_type=jnp.float32).astype(jnp.bfloat16)
    h2 = jnp.maximum(jnp.dot(p, w23, preferred_element_type=jnp.float32),
                     0.0).astype(jnp.bfloat16)
    out_ref[...] = jnp.dot(h2, w4_ref[...].astype(jnp.bfloat16),
                           preferred_element_type=jnp.float32)


def _logits(partial, W2, W3, W4):
    return pl.pallas_call(
        _logits_body,
        out_shape=jax.ShapeDtypeStruct((B, C), jnp.float32),
    )(partial, W2, W3, W4)


def kernel(X, ppr_scores, ppr_idx, W1, W2, W3, W4):
    E = _embed(X, ppr_scores, W1)
    partial = _sc_segsum(E, ppr_idx)
    return _logits(partial, W2, W3, W4)


# speedup vs baseline: 6.8308x; 6.8308x over previous
import jax
import jax.numpy as jnp
from jax import lax
from jax.experimental import pallas as pl
from jax.experimental.pallas import tpu as pltpu

N = 320000
F_IN = 128
H = 128
C = 64
B = 10000

R = 32000
NBLK = N // R
G = 200
NCH = R // G
W = 16
WF = 128
ACC_ROWS = B + 2 * WF


def _body(s0_ref, idx_ref, sc_ref, x_ref, w1_ref, w2_ref, w3_ref, w4_ref,
          out_ref, acc_ref):
    pid = pl.program_id(0)

    @pl.when(pid == 0)
    def _init():
        acc_ref[...] = jnp.zeros((ACC_ROWS, H), jnp.float32)

    x = x_ref[...].astype(jnp.bfloat16)
    h = jnp.maximum(
        jnp.dot(x, w1_ref[...].astype(jnp.bfloat16),
                preferred_element_type=jnp.float32), 0.0
    ).astype(jnp.bfloat16)

    seg = idx_ref[0]
    sc = sc_ref[0]
    iota = lax.broadcasted_iota(jnp.int32, (W, G), 0)

    ov = jnp.zeros((1, G), jnp.int32)
    for c in range(NCH):
        seg_c = seg[:, c * G:(c + 1) * G]
        sc_c = sc[:, c * G:(c + 1) * G]
        s0c = s0_ref[pid * NCH + c]
        local = seg_c - s0c
        oh = jnp.where(local == iota, sc_c, 0.0).astype(jnp.bfloat16)
        contrib = lax.dot_general(oh, h[c * G:(c + 1) * G, :],
                                  (((1,), (0,)), ((), ())),
                                  preferred_element_type=jnp.float32)
        acc_ref[pl.ds(s0c, W), :] += contrib
        ov = jnp.maximum(ov, local)

    @pl.when(jnp.max(ov) >= W)
    def _fallback():
        pieces = []
        for c in range(NCH):
            seg_c = seg[:, c * G:(c + 1) * G]
            sc_c = sc[:, c * G:(c + 1) * G]
            local = seg_c - s0_ref[pid * NCH + c]
            pieces.append(jnp.where(local >= W, sc_c, 0.0))
        scm = jnp.concatenate(pieces, axis=1)
        s0b = s0_ref[pid * NCH]
        localb = seg - s0b
        nwin = jnp.max(localb) // WF + 1
        iota_f = lax.broadcasted_iota(jnp.int32, (WF, R), 0)

        def win(k, carry):
            base = k * WF
            ohf = jnp.where(localb == base + iota_f, scm,
                            0.0).astype(jnp.bfloat16)
            contrib = lax.dot_general(ohf, h, (((1,), (0,)), ((), ())),
                                      preferred_element_type=jnp.float32)
            acc_ref[pl.ds(s0b + base, WF), :] += contrib
            return carry

        lax.fori_loop(0, nwin, win, 0)

    @pl.when(pid == NBLK - 1)
    def _final():
        w23 = jnp.dot(w2_ref[...].astype(jnp.bfloat16),
                      w3_ref[...].astype(jnp.bfloat16),
                      preferred_element_type=jnp.float32).astype(jnp.bfloat16)
        p = acc_ref[0:B, :].astype(jnp.bfloat16)
        h2 = jnp.maximum(
            jnp.dot(p, w23, preferred_element_type=jnp.float32), 0.0
        ).astype(jnp.bfloat16)
        out_ref[...] = jnp.dot(h2, w4_ref[...].astype(jnp.bfloat16),
                               preferred_element_type=jnp.float32)


def kernel(X, ppr_scores, ppr_idx, W1, W2, W3, W4):
    s0s = ppr_idx[::G]
    idx3 = ppr_idx.reshape(NBLK, 1, R)
    sc3 = ppr_scores.reshape(NBLK, 1, R)

    grid_spec = pltpu.PrefetchScalarGridSpec(
        num_scalar_prefetch=1,
        grid=(NBLK,),
        in_specs=[
            pl.BlockSpec((1, 1, R), lambda i, s0s: (i, 0, 0)),
            pl.BlockSpec((1, 1, R), lambda i, s0s: (i, 0, 0)),
            pl.BlockSpec((R, F_IN), lambda i, s0s: (i, 0)),
            pl.BlockSpec((F_IN, H), lambda i, s0s: (0, 0)),
            pl.BlockSpec((H, H), lambda i, s0s: (0, 0)),
            pl.BlockSpec((H, H), lambda i, s0s: (0, 0)),
            pl.BlockSpec((H, C), lambda i, s0s: (0, 0)),
        ],
        out_specs=pl.BlockSpec((B, C), lambda i, s0s: (0, 0)),
        scratch_shapes=[pltpu.VMEM((ACC_ROWS, H), jnp.float32)],
    )

    return pl.pallas_call(
        _body,
        grid_spec=grid_spec,
        out_shape=jax.ShapeDtypeStruct((B, C), jnp.float32),
        compiler_params=pltpu.CompilerParams(
            dimension_semantics=("arbitrary",),
        ),
    )(s0s, idx3, sc3, X, W1, W2, W3, W4)
